# baseline (speedup 1.0000x reference)
import functools
import math

import jax
import jax.numpy as jnp
from jax.experimental import pallas as pl
from jax.experimental.pallas import tpu as pltpu

_BB = 2
_QC = 256


def _attn_kernel(x_ref, w_ref, o_ref, *, dim_attn):
    bb, S, D = x_ref.shape
    a = dim_attn
    qc = _QC

    x2d = x_ref[...].reshape(bb * S, D)
    qkv = jnp.dot(x2d, w_ref[...], preferred_element_type=jnp.float32)

    dnums = (((1,), (1,)), ((), ()))
    for b in range(bb):
        k = qkv[b * S:(b + 1) * S, a:2 * a]
        v = qkv[b * S:(b + 1) * S, 2 * a:]
        for c in range(S // qc):
            qb = qkv[b * S + c * qc:b * S + (c + 1) * qc, :a]
            s = jax.lax.dot_general(qb, k, dnums,
                                    preferred_element_type=jnp.float32)
            m = jnp.max(s, axis=-1, keepdims=True)
            e = jnp.exp(s - m)
            denom = jnp.sum(e, axis=-1, keepdims=True)
            o = jnp.dot(e, v, preferred_element_type=jnp.float32)
            o_ref[b, c * qc:(c + 1) * qc, :] = (
                o * pl.reciprocal(denom, approx=True))


def kernel(x, wq, wk, wv):
    B, S, D = x.shape
    A = wq.shape[1]
    scale = jnp.float32(1.0 / math.sqrt(A))

    wqkv = jnp.concatenate([wq * scale, wk, wv], axis=1)
    x_bf = x

    bb = _BB
    while B % bb:
        bb //= 2

    flops = 2 * B * (S * D * (2 * A + D) + S * S * A + S * S * D)
    bytes_accessed = 4 * (x.size + wqkv.size + B * S * D)

    body = functools.partial(_attn_kernel, dim_attn=A)
    return pl.pallas_call(
        body,
        out_shape=jax.ShapeDtypeStruct((B, S, D), jnp.float32),
        grid=(B // bb,),
        in_specs=[
            pl.BlockSpec((bb, S, D), lambda b: (b, 0, 0)),
            pl.BlockSpec((D, 2 * A + D), lambda b: (0, 0)),
        ],
        out_specs=pl.BlockSpec((bb, S, D), lambda b: (b, 0, 0)),
        compiler_params=pltpu.CompilerParams(
            dimension_semantics=("parallel",)),
        cost_estimate=pl.CostEstimate(
            flops=flops, transcendentals=B * S * S,
            bytes_accessed=bytes_accessed),
    )(x_bf, wqkv)


# speedup vs baseline: 1.4065x; 1.4065x over previous
import functools
import math

import jax
import jax.numpy as jnp
from jax.experimental import pallas as pl
from jax.experimental.pallas import tpu as pltpu

_BB = 2


def _attn_kernel(x_ref, w_ref, o_ref, *, dim_attn):
    bb, S, D = x_ref.shape
    a = dim_attn

    x2d = x_ref[...].reshape(bb * S, D)
    proj = jnp.dot(x2d, w_ref[...], preferred_element_type=jnp.float32)

    qp = proj[:, :a].reshape(bb, S, a)
    v = proj[:, a:].reshape(bb, S, D)

    s = jnp.einsum("bqd,bkd->bqk", qp, x_ref[...],
                   preferred_element_type=jnp.float32)
    m = jnp.max(s, axis=-1, keepdims=True)
    e = jnp.exp(s - m)
    denom = jnp.sum(e, axis=-1, keepdims=True)
    o = jnp.einsum("bqk,bkd->bqd", e, v,
                   preferred_element_type=jnp.float32)
    o_ref[...] = o * pl.reciprocal(denom, approx=True)


def kernel(x, wq, wk, wv):
    B, S, D = x.shape
    A = wq.shape[1]
    scale = jnp.float32(1.0 / math.sqrt(A))

    wqk = jax.lax.dot_general(wq, wk, (((1,), (1,)), ((), ())),
                              precision=jax.lax.Precision.HIGHEST) * scale
    w2 = jnp.concatenate([wqk, wv], axis=1)

    bb = _BB
    while B % bb:
        bb //= 2

    flops = 2 * B * (S * D * (A + D) + S * S * D + S * S * D)
    bytes_accessed = 4 * (x.size + w2.size + B * S * D)

    body = functools.partial(_attn_kernel, dim_attn=A)
    return pl.pallas_call(
        body,
        out_shape=jax.ShapeDtypeStruct((B, S, D), jnp.float32),
        grid=(B // bb,),
        in_specs=[
            pl.BlockSpec((bb, S, D), lambda b: (b, 0, 0)),
            pl.BlockSpec((D, A + D), lambda b: (0, 0)),
        ],
        out_specs=pl.BlockSpec((bb, S, D), lambda b: (b, 0, 0)),
        compiler_params=pltpu.CompilerParams(
            dimension_semantics=("parallel",)),
        cost_estimate=pl.CostEstimate(
            flops=flops, transcendentals=B * S * S,
            bytes_accessed=bytes_accessed),
    )(x, w2)


# speedup vs baseline: 1.4337x; 1.0194x over previous
import functools
import math

import jax
import jax.numpy as jnp
from jax.experimental import pallas as pl
from jax.experimental.pallas import tpu as pltpu

_BB = 2


def _attn_kernel(x_ref, wqk_ref, wv_ref, o_ref):
    bb, S, D = x_ref.shape

    x2d = x_ref[...].reshape(bb * S, D)
    qp = jnp.dot(x2d, wqk_ref[...],
                 preferred_element_type=jnp.float32).reshape(bb, S, D)
    v = jnp.dot(x2d, wv_ref[...],
                preferred_element_type=jnp.float32).reshape(bb, S, D)

    s = jnp.einsum("bqd,bkd->bqk", qp, x_ref[...],
                   preferred_element_type=jnp.float32)
    m = jnp.max(s, axis=-1, keepdims=True)
    e = jnp.exp(s - m)
    denom = jnp.sum(e, axis=-1, keepdims=True)
    o = jnp.einsum("bqk,bkd->bqd", e, v,
                   preferred_element_type=jnp.float32)
    o_ref[...] = o * pl.reciprocal(denom, approx=True)


def kernel(x, wq, wk, wv):
    B, S, D = x.shape
    A = wq.shape[1]
    scale = jnp.float32(1.0 / math.sqrt(A))

    wqk = jax.lax.dot_general(wq, wk, (((1,), (1,)), ((), ())),
                              precision=jax.lax.Precision.HIGHEST) * scale

    bb = _BB
    while B % bb:
        bb //= 2

    flops = 2 * B * (S * D * (A + D) + S * S * D + S * S * D)
    bytes_accessed = 4 * (x.size + wqk.size + wv.size + B * S * D)

    return pl.pallas_call(
        _attn_kernel,
        out_shape=jax.ShapeDtypeStruct((B, S, D), jnp.float32),
        grid=(B // bb,),
        in_specs=[
            pl.BlockSpec((bb, S, D), lambda b: (b, 0, 0)),
            pl.BlockSpec((D, A), lambda b: (0, 0)),
            pl.BlockSpec((D, D), lambda b: (0, 0)),
        ],
        out_specs=pl.BlockSpec((bb, S, D), lambda b: (b, 0, 0)),
        compiler_params=pltpu.CompilerParams(
            dimension_semantics=("parallel",)),
        cost_estimate=pl.CostEstimate(
            flops=flops, transcendentals=B * S * S,
            bytes_accessed=bytes_accessed),
    )(x, wqk, wv)
